# baseline (speedup 1.0000x reference)
import functools

import jax
import jax.numpy as jnp
from jax import lax
from jax.experimental import pallas as pl
from jax.experimental.pallas import tpu as pltpu
from jax.experimental.pallas import tpu_sc as plsc

EMBED = 128
N_ROWS = 4096 * 200
ROW_GROUPS = N_ROWS // EMBED
NC, NS = 2, 16
NW = NC * NS

SC_RG = 4096
TC_RG = ROW_GROUPS - SC_RG
PER_W = SC_RG // NW
QTR = PER_W // 4
NSLOT = 4
NSTEP = PER_W // NSLOT
STEPS_Q = NSTEP // 4
SPQ = QTR // STEPS_Q

BG = 16
TC_GRID = TC_RG // BG


def _tab_body(we_ref, wp_ref, wr_ref, out_ref):
    c = lax.broadcasted_iota(jnp.int32, (48, EMBED), 0)
    n = c % 5
    p = (c // 5) % 3
    r = c // 15
    t = jnp.zeros((48, EMBED), jnp.float32)
    for i in range(5):
        t = t + jnp.where(n == i, we_ref[i, :][None, :], 0.0)
    for i in range(3):
        t = t + jnp.where(p == i, wp_ref[i, :][None, :], 0.0)
    for i in range(3):
        t = t + jnp.where(r == i, wr_ref[i, :][None, :], 0.0)
    out_ref[...] = t


def _combined_table(We, Wpbs, Wrt):
    return pl.pallas_call(
        _tab_body,
        out_shape=jax.ShapeDtypeStruct((48, EMBED), jnp.float32),
    )(We, Wpbs, Wrt)


def _sc_embed(tab_hbm, xn_hbm, xp_hbm, xr_hbm, out_hbm,
              tab_sh, xc_v, xn_v, xp_v, xr_v,
              rows0, rows1, rows2, rows3,
              sg0, sg1, sg2, sg3, sw0, sw1, sw2, sw3, si):
    wid = lax.axis_index("s") * NC + lax.axis_index("c")
    base = wid * PER_W
    rows = (rows0, rows1, rows2, rows3)
    sg = (sg0, sg1, sg2, sg3)
    sw = (sw0, sw1, sw2, sw3)

    @pl.when(lax.axis_index("s") == 0)
    def _stage_tab():
        pltpu.sync_copy(tab_hbm, tab_sh)

    plsc.subcore_barrier()

    def slab_copies(q):
        sl = pl.ds(base + q * QTR, QTR)
        return [
            pltpu.make_async_copy(xn_hbm.at[sl], xn_v, si),
            pltpu.make_async_copy(xp_hbm.at[sl], xp_v, si),
            pltpu.make_async_copy(xr_hbm.at[sl], xr_v, si),
        ]

    def combine_one(src, dst):
        for k in range(EMBED // 16):
            s = pl.ds(k * 16, 16)
            xc_v[dst, 0, s] = (
                xn_v[src, 0, s] + xp_v[src, 0, s] * 5 + xr_v[src, 0, s] * 15
            )

    def gather_chunk(i, slot):
        return pltpu.make_async_copy(tab_sh.at[xc_v.at[i, 0]], rows[slot], sg[slot])

    def write_chunk(i, slot):
        return pltpu.make_async_copy(
            rows[slot], out_hbm.at[pl.ds((base + i) * EMBED, EMBED)], sw[slot]
        )

    for cp in slab_copies(0):
        cp.start()
    for cp in slab_copies(0):
        cp.wait()

    def comb0(j, c):
        combine_one(j, j)
        return c

    lax.fori_loop(0, QTR, comb0, 0)

    for cp in slab_copies(1):
        cp.start()

    for s in range(NSLOT):
        gather_chunk(s, s).start()

    for q in range(4):
        def step(tt, c, q=q):
            t = q * STEPS_Q + tt

            if q < 3:
                @pl.when(tt == 0)
                def _slab_arrived():
                    for cp in slab_copies(q + 1):
                        cp.wait()

                def one(k, cc):
                    combine_one(SPQ * tt + k, SPQ * t + QTR + k)
                    return cc

                lax.fori_loop(0, SPQ, one, 0)

                if q < 2:
                    @pl.when(tt == STEPS_Q - 1)
                    def _slab_next():
                        for cp in slab_copies(q + 2):
                            cp.start()

            for s in range(NSLOT):
                i = NSLOT * t + s
                gather_chunk(i, s).wait()
                write_chunk(i, s).start()

            if q < 3:
                for s in range(NSLOT):
                    i = NSLOT * t + s
                    write_chunk(i, s).wait()
                    gather_chunk(i + NSLOT, s).start()
            else:
                @pl.when(tt < STEPS_Q - 1)
                def _prefetch():
                    for s in range(NSLOT):
                        i = NSLOT * t + s
                        write_chunk(i, s).wait()
                        gather_chunk(i + NSLOT, s).start()

            return c

        lax.fori_loop(0, STEPS_Q, step, 0)

    for s in range(NSLOT):
        write_chunk(PER_W - NSLOT + s, s).wait()


_sc_embed_call = functools.partial(
    pl.kernel,
    out_type=jax.ShapeDtypeStruct((SC_RG * EMBED, EMBED), jnp.float32),
    mesh=plsc.VectorSubcoreMesh(core_axis_name="c", subcore_axis_name="s"),
    scratch_types=[
        pltpu.VMEM_SHARED((48, EMBED), jnp.float32),
        pltpu.VMEM((PER_W, 1, EMBED), jnp.int32),
        pltpu.VMEM((QTR, 1, EMBED), jnp.int32),
        pltpu.VMEM((QTR, 1, EMBED), jnp.int32),
        pltpu.VMEM((QTR, 1, EMBED), jnp.int32),
        pltpu.VMEM((EMBED, EMBED), jnp.float32),
        pltpu.VMEM((EMBED, EMBED), jnp.float32),
        pltpu.VMEM((EMBED, EMBED), jnp.float32),
        pltpu.VMEM((EMBED, EMBED), jnp.float32),
        pltpu.SemaphoreType.DMA,
        pltpu.SemaphoreType.DMA,
        pltpu.SemaphoreType.DMA,
        pltpu.SemaphoreType.DMA,
        pltpu.SemaphoreType.DMA,
        pltpu.SemaphoreType.DMA,
        pltpu.SemaphoreType.DMA,
        pltpu.SemaphoreType.DMA,
        pltpu.SemaphoreType.DMA,
    ],
)(_sc_embed)


def _tc_body(xn_ref, xp_ref, xr_ref, tab_ref, out_ref):
    xc = xn_ref[...] + xp_ref[...] * 5 + xr_ref[...] * 15
    v = lax.broadcasted_iota(jnp.int32, (1, 48, 1), 1)
    oh = (xc[:, None, :] == v).astype(jnp.float32)
    tab3 = jnp.broadcast_to(tab_ref[...][None], (BG, 48, EMBED))
    out_ref[...] = lax.dot_general(
        oh, tab3,
        dimension_numbers=(((1,), (1,)), ((0,), (0,))),
        preferred_element_type=jnp.float32,
    )


def _tc_embed(xn, xp, xr, tab):
    return pl.pallas_call(
        _tc_body,
        grid=(TC_GRID,),
        in_specs=[
            pl.BlockSpec((BG, EMBED), lambda i: (i, 0)),
            pl.BlockSpec((BG, EMBED), lambda i: (i, 0)),
            pl.BlockSpec((BG, EMBED), lambda i: (i, 0)),
            pl.BlockSpec((48, EMBED), lambda i: (0, 0)),
        ],
        out_specs=pl.BlockSpec((BG, EMBED, EMBED), lambda i: (i, 0, 0)),
        out_shape=jax.ShapeDtypeStruct((TC_RG, EMBED, EMBED), jnp.float32),
    )(xn, xp, xr, tab)


@jax.jit
def kernel(X_nucl, X_pbs, X_rt, We, Wpbs, Wrt):
    xn = X_nucl.astype(jnp.int32).reshape(ROW_GROUPS, EMBED)
    xp = X_pbs.astype(jnp.int32).reshape(ROW_GROUPS, EMBED)
    xr = X_rt.astype(jnp.int32).reshape(ROW_GROUPS, EMBED)
    tab = _combined_table(We, Wpbs, Wrt)
    sc_out = _sc_embed_call(
        tab,
        xn[:SC_RG].reshape(SC_RG, 1, EMBED),
        xp[:SC_RG].reshape(SC_RG, 1, EMBED),
        xr[:SC_RG].reshape(SC_RG, 1, EMBED),
    )
    tc_out = _tc_embed(xn[SC_RG:], xp[SC_RG:], xr[SC_RG:], tab)
    out = jnp.concatenate(
        [sc_out, tc_out.reshape(TC_RG * EMBED, EMBED)], axis=0
    )
    return out.reshape(X_nucl.shape[0], X_nucl.shape[1], EMBED)
